# baseline (speedup 1.0000x reference)
import jax
import jax.numpy as jnp
from jax import lax
from jax.experimental import pallas as pl
from jax.experimental.pallas import tpu as pltpu
from jax.experimental.pallas import tpu_sc as plsc

N = 10000
D = 256
EH = 5000
NNZ = 160000
HD = 128
NT = 16
C = 128
N_IT = 80
NNZP = NT * N_IT * C
EHP = 5120
NODE_T = 632
NP = NT * NODE_T
ZR = 40
BN = 1264
BE = 632


def _zero_vmem(zb, nrows):
    def _zero_row(r, _):
        for v in range(HD // 16):
            zb[r, pl.ds(v * 16, 16)] = jnp.zeros((16,), jnp.float32)
        return _
    lax.fori_loop(0, nrows, _zero_row, None)


def _spmm_body(tabl, tabr, gidx, sidx, outl, outr,
               acc, gv2, sv2, rows, sem):
    c = lax.axis_index("c")
    t = lax.axis_index("s")
    zb = rows.at[pl.ds(0, ZR)]
    rem = NODE_T % ZR

    _zero_vmem(zb, ZR)
    for k in range(NODE_T // ZR):
        pltpu.sync_copy(zb, acc.at[pl.ds(t * NODE_T + k * ZR, ZR)])
    pltpu.sync_copy(rows.at[pl.ds(0, rem)],
                    acc.at[pl.ds(t * NODE_T + NODE_T - rem, rem)])

    pltpu.sync_copy(gidx.at[t], gv2)
    pltpu.sync_copy(sidx.at[t], sv2)
    plsc.subcore_barrier()

    def _loop(tab):
        @pl.loop(0, N_IT)
        def _(i):
            pltpu.async_copy(tab.at[gv2.at[i]], rows, sem).wait()
            pltpu.sync_copy(rows, acc.at[sv2.at[i]], add=True)

    @pl.when(c == 0)
    def _():
        _loop(tabl)

    @pl.when(c == 1)
    def _():
        _loop(tabr)

    plsc.subcore_barrier()

    def _wout(k, _):
        zb = rows.at[pl.ds(0, ZR)]
        pltpu.sync_copy(acc.at[pl.ds(t * NODE_T + k * ZR, ZR)], zb)

        @pl.when(c == 0)
        def _():
            pltpu.sync_copy(zb, outl.at[pl.ds(t * NODE_T + k * ZR, ZR)])

        @pl.when(c == 1)
        def _():
            pltpu.sync_copy(zb, outr.at[pl.ds(t * NODE_T + k * ZR, ZR)])
        return _
    lax.fori_loop(0, NODE_T // ZR, _wout, None)

    pltpu.sync_copy(acc.at[pl.ds(t * NODE_T + NODE_T - rem, rem)],
                    rows.at[pl.ds(0, rem)])

    @pl.when(c == 0)
    def _():
        pltpu.sync_copy(rows.at[pl.ds(0, rem)],
                        outl.at[pl.ds(t * NODE_T + NODE_T - rem, rem)])

    @pl.when(c == 1)
    def _():
        pltpu.sync_copy(rows.at[pl.ds(0, rem)],
                        outr.at[pl.ds(t * NODE_T + NODE_T - rem, rem)])


_sc_spmm = pl.kernel(
    _spmm_body,
    out_type=[jax.ShapeDtypeStruct((NP, HD), jnp.float32),
              jax.ShapeDtypeStruct((NP, HD), jnp.float32)],
    mesh=plsc.VectorSubcoreMesh(core_axis_name="c", subcore_axis_name="s"),
    scratch_types=[
        pltpu.VMEM_SHARED((NP, HD), jnp.float32),
        pltpu.VMEM((N_IT, C), jnp.int32),
        pltpu.VMEM((N_IT, C), jnp.int32),
        pltpu.VMEM((C, HD), jnp.float32),
        pltpu.SemaphoreType.DMA,
    ],
)


def _pre_body(x_ref, dv_ref, l_ref, r_ref):
    xn = x_ref[...] * dv_ref[...]
    l_ref[...] = xn[:, :HD]
    r_ref[...] = xn[:, HD:]


_tc_pre = pl.pallas_call(
    _pre_body,
    grid=(NP // BN,),
    in_specs=[pl.BlockSpec((BN, D), lambda i: (i, 0)),
              pl.BlockSpec((BN, 1), lambda i: (i, 0))],
    out_specs=[pl.BlockSpec((BN, HD), lambda i: (i, 0)),
               pl.BlockSpec((BN, HD), lambda i: (i, 0))],
    out_shape=[jax.ShapeDtypeStruct((NP, HD), jnp.float32),
               jax.ShapeDtypeStruct((NP, HD), jnp.float32)],
)


def _merge_body(a_ref, b_ref, de_ref, l_ref, r_ref):
    de = de_ref[...]
    l_ref[...] = a_ref[...] * de
    r_ref[...] = b_ref[...] * de


_tc_merge = pl.pallas_call(
    _merge_body,
    grid=(NP // BE,),
    in_specs=[pl.BlockSpec((BE, HD), lambda i: (i, 0)),
              pl.BlockSpec((BE, HD), lambda i: (i, 0)),
              pl.BlockSpec((BE, 1), lambda i: (i, 0))],
    out_specs=[pl.BlockSpec((BE, HD), lambda i: (i, 0)),
               pl.BlockSpec((BE, HD), lambda i: (i, 0))],
    out_shape=[jax.ShapeDtypeStruct((NP, HD), jnp.float32),
               jax.ShapeDtypeStruct((NP, HD), jnp.float32)],
)


def _post_body(l_ref, r_ref, dv_ref, res_ref, w_ref, b_ref, g_ref, be_ref,
               xh_ref, xnl_ref, xnr_ref):
    dv = dv_ref[...]
    xo = jnp.concatenate([l_ref[...], r_ref[...]], axis=1) * dv
    y = jnp.dot(xo, w_ref[...], preferred_element_type=jnp.float32)
    y = y + b_ref[...] + res_ref[...]
    mu = jnp.mean(y, axis=1, keepdims=True)
    yc = y - mu
    var = jnp.mean(yc * yc, axis=1, keepdims=True)
    z = yc * lax.rsqrt(var + 1e-5) * g_ref[...] + be_ref[...]
    xh = jnp.maximum(z, 0.0)
    xh_ref[...] = xh
    xn = xh * dv
    xnl_ref[...] = xn[:, :HD]
    xnr_ref[...] = xn[:, HD:]


_tc_post = pl.pallas_call(
    _post_body,
    grid=(NP // BN,),
    in_specs=[pl.BlockSpec((BN, HD), lambda i: (i, 0)),
              pl.BlockSpec((BN, HD), lambda i: (i, 0)),
              pl.BlockSpec((BN, 1), lambda i: (i, 0)),
              pl.BlockSpec((BN, D), lambda i: (i, 0)),
              pl.BlockSpec((D, D), lambda i: (0, 0)),
              pl.BlockSpec((1, D), lambda i: (0, 0)),
              pl.BlockSpec((1, D), lambda i: (0, 0)),
              pl.BlockSpec((1, D), lambda i: (0, 0))],
    out_specs=[pl.BlockSpec((BN, D), lambda i: (i, 0)),
               pl.BlockSpec((BN, HD), lambda i: (i, 0)),
               pl.BlockSpec((BN, HD), lambda i: (i, 0))],
    out_shape=[jax.ShapeDtypeStruct((NP, D), jnp.float32),
               jax.ShapeDtypeStruct((NP, HD), jnp.float32),
               jax.ShapeDtypeStruct((NP, HD), jnp.float32)],
)


def _fuse_body(x1_ref, x2_ref, wa_ref, ba_ref, o_ref):
    a = x1_ref[...]
    b = x2_ref[...]
    wv = wa_ref[...]
    s1 = jnp.dot(a, wv, preferred_element_type=jnp.float32) + ba_ref[...]
    s2 = jnp.dot(b, wv, preferred_element_type=jnp.float32) + ba_ref[...]
    m = jnp.maximum(s1, s2)
    e1 = jnp.exp(s1 - m)
    e2 = jnp.exp(s2 - m)
    w1 = e1 / (e1 + e2)
    o_ref[...] = w1 * a + (1.0 - w1) * b


_tc_fuse = pl.pallas_call(
    _fuse_body,
    grid=(NP // BN,),
    in_specs=[pl.BlockSpec((BN, D), lambda i: (i, 0)),
              pl.BlockSpec((BN, D), lambda i: (i, 0)),
              pl.BlockSpec((D, 1), lambda i: (0, 0)),
              pl.BlockSpec((1, 1), lambda i: (0, 0))],
    out_specs=pl.BlockSpec((BN, D), lambda i: (i, 0)),
    out_shape=jax.ShapeDtypeStruct((NP, D), jnp.float32),
)


def kernel(X, h1_node_idx, h1_edge_idx, h1_DV_inv_sqrt, h1_DE_inv,
           h2_node_idx, h2_edge_idx, h2_DV_inv_sqrt, h2_DE_inv,
           W1, b1, W2, b2, g1, beta1, g2, beta2, Wa, ba):
    params = [(W1, b1.reshape(1, D), g1.reshape(1, D), beta1.reshape(1, D)),
              (W2, b2.reshape(1, D), g2.reshape(1, D), beta2.reshape(1, D))]
    Xp = jnp.pad(X, ((0, NP - N), (0, 0)))

    def branch(nidx, eidx, dv, de):
        nidx = jnp.concatenate(
            [nidx.astype(jnp.int32), jnp.zeros((NNZP - NNZ,), jnp.int32)]
        ).reshape(NT, N_IT, C)
        eidx = jnp.concatenate(
            [eidx.astype(jnp.int32), jnp.full((NNZP - NNZ,), EHP - 1, jnp.int32)]
        ).reshape(NT, N_IT, C)
        dv2 = jnp.pad(dv, (0, NP - N)).reshape(NP, 1)
        dep2 = jnp.pad(de, (0, NP - EH)).reshape(NP, 1)
        xh = Xp
        xnl, xnr = _tc_pre(Xp, dv2)
        for w, bb, gg, be in params:
            hxl, hxr = _sc_spmm(xnl, xnr, nidx, eidx)
            hxsl, hxsr = _tc_merge(hxl, hxr, dep2)
            ol, orr = _sc_spmm(hxsl, hxsr, eidx, nidx)
            xh, xnl, xnr = _tc_post(ol, orr, dv2, xh, w, bb, gg, be)
        return xh

    X1 = branch(h1_node_idx, h1_edge_idx, h1_DV_inv_sqrt, h1_DE_inv)
    X2 = branch(h2_node_idx, h2_edge_idx, h2_DV_inv_sqrt, h2_DE_inv)
    return _tc_fuse(X1, X2, Wa, ba.reshape(1, 1))[:N]


# speedup vs baseline: 1.1307x; 1.1307x over previous
import jax
import jax.numpy as jnp
from jax import lax
from jax.experimental import pallas as pl
from jax.experimental.pallas import tpu as pltpu
from jax.experimental.pallas import tpu_sc as plsc

N = 10000
D = 256
EH = 5000
NNZ = 160000
HD = 128
NT = 16
C = 128
N_IT = 80
NNZP = NT * N_IT * C
EHP = 5120
NODE_T = 632
NP = NT * NODE_T
ZR = 40
BN = 1264
BE = 632


def _zero_vmem(zb, nrows):
    def _zero_row(r, _):
        for v in range(HD // 16):
            zb[r, pl.ds(v * 16, 16)] = jnp.zeros((16,), jnp.float32)
        return _
    lax.fori_loop(0, nrows, _zero_row, None)


def _spmm_body(tabl, tabr, gidx, sidx, outl, outr,
               acc, gvr, svr, rows0, rows1, gsem0, gsem1, isem):
    c = lax.axis_index("c")
    t = lax.axis_index("s")
    rows = rows0
    zb = rows.at[pl.ds(0, ZR)]
    rem = NODE_T % ZR

    _zero_vmem(zb, ZR)
    for k in range(NODE_T // ZR):
        pltpu.sync_copy(zb, acc.at[pl.ds(t * NODE_T + k * ZR, ZR)])
    pltpu.sync_copy(rows.at[pl.ds(0, rem)],
                    acc.at[pl.ds(t * NODE_T + NODE_T - rem, rem)])
    plsc.subcore_barrier()

    def _issue_idx(i, s):
        pltpu.async_copy(gidx.at[t, i], gvr.at[s], isem)
        pltpu.async_copy(sidx.at[t, i], svr.at[s], isem)

    def _wait_idx(i, s):
        pltpu.make_async_copy(gidx.at[t, i], gvr.at[s], isem).wait()
        pltpu.make_async_copy(sidx.at[t, i], svr.at[s], isem).wait()

    def _loop(tab):
        rb = [rows0, rows1]
        gs = [gsem0, gsem1]
        _wait_idx(0, 0)
        pltpu.async_copy(tab.at[gvr.at[0]], rows0, gsem0)
        _issue_idx(1, 1)

        @pl.loop(0, N_IT, step=2)
        def _(i):
            for u in range(2):
                s, o = u, 1 - u
                idx = i + u
                pltpu.make_async_copy(tab.at[gvr.at[s]], rb[s], gs[s]).wait()

                @pl.when(idx + 1 < N_IT)
                def _(idx=idx, s=s, o=o):
                    _wait_idx(idx + 1, o)
                    pltpu.async_copy(tab.at[gvr.at[o]], rb[o], gs[o])

                pltpu.sync_copy(rb[s], acc.at[svr.at[s]], add=True)

                @pl.when(idx + 2 < N_IT)
                def _(idx=idx, s=s):
                    _issue_idx(idx + 2, s)

    _issue_idx(0, 0)

    @pl.when(c == 0)
    def _():
        _loop(tabl)

    @pl.when(c == 1)
    def _():
        _loop(tabr)

    plsc.subcore_barrier()

    def _wout(k, _):
        zb = rows0.at[pl.ds(0, ZR)]
        pltpu.sync_copy(acc.at[pl.ds(t * NODE_T + k * ZR, ZR)], zb)

        @pl.when(c == 0)
        def _():
            pltpu.sync_copy(zb, outl.at[pl.ds(t * NODE_T + k * ZR, ZR)])

        @pl.when(c == 1)
        def _():
            pltpu.sync_copy(zb, outr.at[pl.ds(t * NODE_T + k * ZR, ZR)])
        return _
    lax.fori_loop(0, NODE_T // ZR, _wout, None)

    pltpu.sync_copy(acc.at[pl.ds(t * NODE_T + NODE_T - rem, rem)],
                    rows0.at[pl.ds(0, rem)])

    @pl.when(c == 0)
    def _():
        pltpu.sync_copy(rows0.at[pl.ds(0, rem)],
                        outl.at[pl.ds(t * NODE_T + NODE_T - rem, rem)])

    @pl.when(c == 1)
    def _():
        pltpu.sync_copy(rows0.at[pl.ds(0, rem)],
                        outr.at[pl.ds(t * NODE_T + NODE_T - rem, rem)])


_sc_spmm = pl.kernel(
    _spmm_body,
    out_type=[jax.ShapeDtypeStruct((NP, HD), jnp.float32),
              jax.ShapeDtypeStruct((NP, HD), jnp.float32)],
    mesh=plsc.VectorSubcoreMesh(core_axis_name="c", subcore_axis_name="s"),
    scratch_types=[
        pltpu.VMEM_SHARED((NP, HD), jnp.float32),
        pltpu.VMEM((2, C), jnp.int32),
        pltpu.VMEM((2, C), jnp.int32),
        pltpu.VMEM((C, HD), jnp.float32),
        pltpu.VMEM((C, HD), jnp.float32),
        pltpu.SemaphoreType.DMA,
        pltpu.SemaphoreType.DMA,
        pltpu.SemaphoreType.DMA,
    ],
)


def _pre_body(x_ref, dv_ref, l_ref, r_ref):
    xn = x_ref[...] * dv_ref[...]
    l_ref[...] = xn[:, :HD]
    r_ref[...] = xn[:, HD:]


_tc_pre = pl.pallas_call(
    _pre_body,
    grid=(NP // BN,),
    in_specs=[pl.BlockSpec((BN, D), lambda i: (i, 0)),
              pl.BlockSpec((BN, 1), lambda i: (i, 0))],
    out_specs=[pl.BlockSpec((BN, HD), lambda i: (i, 0)),
               pl.BlockSpec((BN, HD), lambda i: (i, 0))],
    out_shape=[jax.ShapeDtypeStruct((NP, HD), jnp.float32),
               jax.ShapeDtypeStruct((NP, HD), jnp.float32)],
)


def _merge_body(a_ref, b_ref, de_ref, l_ref, r_ref):
    de = de_ref[...]
    l_ref[...] = a_ref[...] * de
    r_ref[...] = b_ref[...] * de


_tc_merge = pl.pallas_call(
    _merge_body,
    grid=(NP // BE,),
    in_specs=[pl.BlockSpec((BE, HD), lambda i: (i, 0)),
              pl.BlockSpec((BE, HD), lambda i: (i, 0)),
              pl.BlockSpec((BE, 1), lambda i: (i, 0))],
    out_specs=[pl.BlockSpec((BE, HD), lambda i: (i, 0)),
               pl.BlockSpec((BE, HD), lambda i: (i, 0))],
    out_shape=[jax.ShapeDtypeStruct((NP, HD), jnp.float32),
               jax.ShapeDtypeStruct((NP, HD), jnp.float32)],
)


def _post_body(l_ref, r_ref, dv_ref, res_ref, w_ref, b_ref, g_ref, be_ref,
               xh_ref, xnl_ref, xnr_ref):
    dv = dv_ref[...]
    xo = jnp.concatenate([l_ref[...], r_ref[...]], axis=1) * dv
    y = jnp.dot(xo, w_ref[...], preferred_element_type=jnp.float32)
    y = y + b_ref[...] + res_ref[...]
    mu = jnp.mean(y, axis=1, keepdims=True)
    yc = y - mu
    var = jnp.mean(yc * yc, axis=1, keepdims=True)
    z = yc * lax.rsqrt(var + 1e-5) * g_ref[...] + be_ref[...]
    xh = jnp.maximum(z, 0.0)
    xh_ref[...] = xh
    xn = xh * dv
    xnl_ref[...] = xn[:, :HD]
    xnr_ref[...] = xn[:, HD:]


_tc_post = pl.pallas_call(
    _post_body,
    grid=(NP // BN,),
    in_specs=[pl.BlockSpec((BN, HD), lambda i: (i, 0)),
              pl.BlockSpec((BN, HD), lambda i: (i, 0)),
              pl.BlockSpec((BN, 1), lambda i: (i, 0)),
              pl.BlockSpec((BN, D), lambda i: (i, 0)),
              pl.BlockSpec((D, D), lambda i: (0, 0)),
              pl.BlockSpec((1, D), lambda i: (0, 0)),
              pl.BlockSpec((1, D), lambda i: (0, 0)),
              pl.BlockSpec((1, D), lambda i: (0, 0))],
    out_specs=[pl.BlockSpec((BN, D), lambda i: (i, 0)),
               pl.BlockSpec((BN, HD), lambda i: (i, 0)),
               pl.BlockSpec((BN, HD), lambda i: (i, 0))],
    out_shape=[jax.ShapeDtypeStruct((NP, D), jnp.float32),
               jax.ShapeDtypeStruct((NP, HD), jnp.float32),
               jax.ShapeDtypeStruct((NP, HD), jnp.float32)],
)


def _fuse_body(x1_ref, x2_ref, wa_ref, ba_ref, o_ref):
    a = x1_ref[...]
    b = x2_ref[...]
    wv = wa_ref[...]
    s1 = jnp.dot(a, wv, preferred_element_type=jnp.float32) + ba_ref[...]
    s2 = jnp.dot(b, wv, preferred_element_type=jnp.float32) + ba_ref[...]
    m = jnp.maximum(s1, s2)
    e1 = jnp.exp(s1 - m)
    e2 = jnp.exp(s2 - m)
    w1 = e1 / (e1 + e2)
    o_ref[...] = w1 * a + (1.0 - w1) * b


_tc_fuse = pl.pallas_call(
    _fuse_body,
    grid=(NP // BN,),
    in_specs=[pl.BlockSpec((BN, D), lambda i: (i, 0)),
              pl.BlockSpec((BN, D), lambda i: (i, 0)),
              pl.BlockSpec((D, 1), lambda i: (0, 0)),
              pl.BlockSpec((1, 1), lambda i: (0, 0))],
    out_specs=pl.BlockSpec((BN, D), lambda i: (i, 0)),
    out_shape=jax.ShapeDtypeStruct((NP, D), jnp.float32),
)


def kernel(X, h1_node_idx, h1_edge_idx, h1_DV_inv_sqrt, h1_DE_inv,
           h2_node_idx, h2_edge_idx, h2_DV_inv_sqrt, h2_DE_inv,
           W1, b1, W2, b2, g1, beta1, g2, beta2, Wa, ba):
    params = [(W1, b1.reshape(1, D), g1.reshape(1, D), beta1.reshape(1, D)),
              (W2, b2.reshape(1, D), g2.reshape(1, D), beta2.reshape(1, D))]
    Xp = jnp.pad(X, ((0, NP - N), (0, 0)))

    def branch(nidx, eidx, dv, de):
        nidx = jnp.concatenate(
            [nidx.astype(jnp.int32), jnp.zeros((NNZP - NNZ,), jnp.int32)]
        ).reshape(NT, N_IT, C)
        eidx = jnp.concatenate(
            [eidx.astype(jnp.int32), jnp.full((NNZP - NNZ,), EHP - 1, jnp.int32)]
        ).reshape(NT, N_IT, C)
        dv2 = jnp.pad(dv, (0, NP - N)).reshape(NP, 1)
        dep2 = jnp.pad(de, (0, NP - EH)).reshape(NP, 1)
        xh = Xp
        xnl, xnr = _tc_pre(Xp, dv2)
        for w, bb, gg, be in params:
            hxl, hxr = _sc_spmm(xnl, xnr, nidx, eidx)
            hxsl, hxsr = _tc_merge(hxl, hxr, dep2)
            ol, orr = _sc_spmm(hxsl, hxsr, eidx, nidx)
            xh, xnl, xnr = _tc_post(ol, orr, dv2, xh, w, bb, gg, be)
        return xh

    X1 = branch(h1_node_idx, h1_edge_idx, h1_DV_inv_sqrt, h1_DE_inv)
    X2 = branch(h2_node_idx, h2_edge_idx, h2_DV_inv_sqrt, h2_DE_inv)
    return _tc_fuse(X1, X2, Wa, ba.reshape(1, 1))[:N]


# speedup vs baseline: 1.2020x; 1.0630x over previous
import jax
import jax.numpy as jnp
from jax import lax
from jax.experimental import pallas as pl
from jax.experimental.pallas import tpu as pltpu
from jax.experimental.pallas import tpu_sc as plsc

N = 10000
D = 256
EH = 5000
NNZ = 160000
HD = 128
NT = 16
C = 128
N_IT = 80
NNZP = NT * N_IT * C
EHP = 5120
NODE_T = 632
NP = NT * NODE_T
ZR = 40
BN = 1264
BE = 632


def _zero_vmem(zb, nrows):
    def _zero_row(r, _):
        for v in range(HD // 16):
            zb[r, pl.ds(v * 16, 16)] = jnp.zeros((16,), jnp.float32)
        return _
    lax.fori_loop(0, nrows, _zero_row, None)


def _spmm_body(tabl, tabr, idx3, outl, outr,
               acc, islab, rows0, rows1,
               gsem0, gsem1, ssem, isem0, isem1, isem2, isem3):
    c = lax.axis_index("c")
    t = lax.axis_index("s")
    zb = rows0.at[pl.ds(0, ZR)]
    rem = NODE_T % ZR

    _zero_vmem(zb, ZR)
    for k in range(NODE_T // ZR):
        pltpu.sync_copy(zb, acc.at[pl.ds(t * NODE_T + k * ZR, ZR)])
    pltpu.sync_copy(rows0.at[pl.ds(0, rem)],
                    acc.at[pl.ds(t * NODE_T + NODE_T - rem, rem)])

    isems = [isem0, isem1, isem2, isem3]

    def _issue_islab(i, q):
        pltpu.async_copy(idx3.at[t, i], islab.at[q], isems[q])

    def _wait_islab(i, q):
        pltpu.make_async_copy(idx3.at[t, i], islab.at[q], isems[q]).wait()

    _issue_islab(0, 0)
    plsc.subcore_barrier()

    def _loop(tab):
        rb = [rows0, rows1]
        gs = [gsem0, gsem1]
        _wait_islab(0, 0)
        pltpu.async_copy(tab.at[islab.at[0, 0]], rows0, gsem0)
        _issue_islab(1, 1)
        _issue_islab(2, 2)

        @pl.loop(0, N_IT, step=4)
        def _(i):
            for u in range(4):
                s, o = u % 2, 1 - u % 2
                q, q1 = u % 4, (u + 1) % 4
                idx = i + u
                pltpu.make_async_copy(tab.at[islab.at[q, 0]], rb[s], gs[s]).wait()
                if u == 0:
                    @pl.when(i > 0)
                    def _(o=o):
                        pltpu.make_async_copy(
                            rb[o], acc.at[islab.at[3, 1]], ssem).wait()
                else:
                    pltpu.make_async_copy(
                        rb[o], acc.at[islab.at[q - 1, 1]], ssem).wait()
                if u < 3:
                    _wait_islab(idx + 1, q1)
                    pltpu.async_copy(tab.at[islab.at[q1, 0]], rb[o], gs[o])
                else:
                    @pl.when(idx + 1 < N_IT)
                    def _(idx=idx, o=o, q1=q1):
                        _wait_islab(idx + 1, q1)
                        pltpu.async_copy(tab.at[islab.at[q1, 0]], rb[o], gs[o])
                pltpu.async_copy(rb[s], acc.at[islab.at[q, 1]], ssem, add=True)
                @pl.when(idx + 3 < N_IT)
                def _(idx=idx, q=q):
                    _issue_islab(idx + 3, (q + 3) % 4)

        pltpu.make_async_copy(
            rb[(N_IT - 1) % 2], acc.at[islab.at[(N_IT - 1) % 4, 1]], ssem).wait()

    @pl.when(c == 0)
    def _():
        _loop(tabl)

    @pl.when(c == 1)
    def _():
        _loop(tabr)

    plsc.subcore_barrier()

    def _wout(k, _):
        zb = rows0.at[pl.ds(0, ZR)]
        pltpu.sync_copy(acc.at[pl.ds(t * NODE_T + k * ZR, ZR)], zb)

        @pl.when(c == 0)
        def _():
            pltpu.sync_copy(zb, outl.at[pl.ds(t * NODE_T + k * ZR, ZR)])

        @pl.when(c == 1)
        def _():
            pltpu.sync_copy(zb, outr.at[pl.ds(t * NODE_T + k * ZR, ZR)])
        return _
    lax.fori_loop(0, NODE_T // ZR, _wout, None)

    pltpu.sync_copy(acc.at[pl.ds(t * NODE_T + NODE_T - rem, rem)],
                    rows0.at[pl.ds(0, rem)])

    @pl.when(c == 0)
    def _():
        pltpu.sync_copy(rows0.at[pl.ds(0, rem)],
                        outl.at[pl.ds(t * NODE_T + NODE_T - rem, rem)])

    @pl.when(c == 1)
    def _():
        pltpu.sync_copy(rows0.at[pl.ds(0, rem)],
                        outr.at[pl.ds(t * NODE_T + NODE_T - rem, rem)])


_sc_spmm = pl.kernel(
    _spmm_body,
    out_type=[jax.ShapeDtypeStruct((NP, HD), jnp.float32),
              jax.ShapeDtypeStruct((NP, HD), jnp.float32)],
    mesh=plsc.VectorSubcoreMesh(core_axis_name="c", subcore_axis_name="s"),
    scratch_types=[
        pltpu.VMEM_SHARED((NP, HD), jnp.float32),
        pltpu.VMEM((4, 2, C), jnp.int32),
        pltpu.VMEM((C, HD), jnp.float32),
        pltpu.VMEM((C, HD), jnp.float32),
        pltpu.SemaphoreType.DMA,
        pltpu.SemaphoreType.DMA,
        pltpu.SemaphoreType.DMA,
        pltpu.SemaphoreType.DMA,
        pltpu.SemaphoreType.DMA,
        pltpu.SemaphoreType.DMA,
        pltpu.SemaphoreType.DMA,
    ],
)


def _pre_body(x_ref, dv_ref, l_ref, r_ref):
    xn = x_ref[...] * dv_ref[...]
    l_ref[...] = xn[:, :HD]
    r_ref[...] = xn[:, HD:]


_tc_pre = pl.pallas_call(
    _pre_body,
    grid=(NP // BN,),
    in_specs=[pl.BlockSpec((BN, D), lambda i: (i, 0)),
              pl.BlockSpec((BN, 1), lambda i: (i, 0))],
    out_specs=[pl.BlockSpec((BN, HD), lambda i: (i, 0)),
               pl.BlockSpec((BN, HD), lambda i: (i, 0))],
    out_shape=[jax.ShapeDtypeStruct((NP, HD), jnp.float32),
               jax.ShapeDtypeStruct((NP, HD), jnp.float32)],
)


def _merge_body(a_ref, b_ref, de_ref, l_ref, r_ref):
    de = de_ref[...]
    l_ref[...] = a_ref[...] * de
    r_ref[...] = b_ref[...] * de


_tc_merge = pl.pallas_call(
    _merge_body,
    grid=(NP // BE,),
    in_specs=[pl.BlockSpec((BE, HD), lambda i: (i, 0)),
              pl.BlockSpec((BE, HD), lambda i: (i, 0)),
              pl.BlockSpec((BE, 1), lambda i: (i, 0))],
    out_specs=[pl.BlockSpec((BE, HD), lambda i: (i, 0)),
               pl.BlockSpec((BE, HD), lambda i: (i, 0))],
    out_shape=[jax.ShapeDtypeStruct((NP, HD), jnp.float32),
               jax.ShapeDtypeStruct((NP, HD), jnp.float32)],
)


def _post_body(l_ref, r_ref, dv_ref, res_ref, w_ref, b_ref, g_ref, be_ref,
               xh_ref, xnl_ref, xnr_ref):
    dv = dv_ref[...]
    xo = jnp.concatenate([l_ref[...], r_ref[...]], axis=1) * dv
    y = jnp.dot(xo, w_ref[...], preferred_element_type=jnp.float32)
    y = y + b_ref[...] + res_ref[...]
    mu = jnp.mean(y, axis=1, keepdims=True)
    yc = y - mu
    var = jnp.mean(yc * yc, axis=1, keepdims=True)
    z = yc * lax.rsqrt(var + 1e-5) * g_ref[...] + be_ref[...]
    xh = jnp.maximum(z, 0.0)
    xh_ref[...] = xh
    xn = xh * dv
    xnl_ref[...] = xn[:, :HD]
    xnr_ref[...] = xn[:, HD:]


_tc_post = pl.pallas_call(
    _post_body,
    grid=(NP // BN,),
    in_specs=[pl.BlockSpec((BN, HD), lambda i: (i, 0)),
              pl.BlockSpec((BN, HD), lambda i: (i, 0)),
              pl.BlockSpec((BN, 1), lambda i: (i, 0)),
              pl.BlockSpec((BN, D), lambda i: (i, 0)),
              pl.BlockSpec((D, D), lambda i: (0, 0)),
              pl.BlockSpec((1, D), lambda i: (0, 0)),
              pl.BlockSpec((1, D), lambda i: (0, 0)),
              pl.BlockSpec((1, D), lambda i: (0, 0))],
    out_specs=[pl.BlockSpec((BN, D), lambda i: (i, 0)),
               pl.BlockSpec((BN, HD), lambda i: (i, 0)),
               pl.BlockSpec((BN, HD), lambda i: (i, 0))],
    out_shape=[jax.ShapeDtypeStruct((NP, D), jnp.float32),
               jax.ShapeDtypeStruct((NP, HD), jnp.float32),
               jax.ShapeDtypeStruct((NP, HD), jnp.float32)],
)


def _fuse_body(x1_ref, x2_ref, wa_ref, ba_ref, o_ref):
    a = x1_ref[...]
    b = x2_ref[...]
    wv = wa_ref[...]
    s1 = jnp.dot(a, wv, preferred_element_type=jnp.float32) + ba_ref[...]
    s2 = jnp.dot(b, wv, preferred_element_type=jnp.float32) + ba_ref[...]
    m = jnp.maximum(s1, s2)
    e1 = jnp.exp(s1 - m)
    e2 = jnp.exp(s2 - m)
    w1 = e1 / (e1 + e2)
    o_ref[...] = w1 * a + (1.0 - w1) * b


_tc_fuse = pl.pallas_call(
    _fuse_body,
    grid=(NP // BN,),
    in_specs=[pl.BlockSpec((BN, D), lambda i: (i, 0)),
              pl.BlockSpec((BN, D), lambda i: (i, 0)),
              pl.BlockSpec((D, 1), lambda i: (0, 0)),
              pl.BlockSpec((1, 1), lambda i: (0, 0))],
    out_specs=pl.BlockSpec((BN, D), lambda i: (i, 0)),
    out_shape=jax.ShapeDtypeStruct((NP, D), jnp.float32),
)


def kernel(X, h1_node_idx, h1_edge_idx, h1_DV_inv_sqrt, h1_DE_inv,
           h2_node_idx, h2_edge_idx, h2_DV_inv_sqrt, h2_DE_inv,
           W1, b1, W2, b2, g1, beta1, g2, beta2, Wa, ba):
    params = [(W1, b1.reshape(1, D), g1.reshape(1, D), beta1.reshape(1, D)),
              (W2, b2.reshape(1, D), g2.reshape(1, D), beta2.reshape(1, D))]
    Xp = jnp.pad(X, ((0, NP - N), (0, 0)))

    def branch(nidx, eidx, dv, de):
        nidx = jnp.concatenate(
            [nidx.astype(jnp.int32), jnp.zeros((NNZP - NNZ,), jnp.int32)]
        ).reshape(NT, N_IT, C)
        eidx = jnp.concatenate(
            [eidx.astype(jnp.int32), jnp.full((NNZP - NNZ,), EHP - 1, jnp.int32)]
        ).reshape(NT, N_IT, C)
        idx_ne = jnp.stack([nidx, eidx], axis=2)
        idx_en = jnp.stack([eidx, nidx], axis=2)
        dv2 = jnp.pad(dv, (0, NP - N)).reshape(NP, 1)
        dep2 = jnp.pad(de, (0, NP - EH)).reshape(NP, 1)
        xh = Xp
        xnl, xnr = _tc_pre(Xp, dv2)
        for w, bb, gg, be in params:
            hxl, hxr = _sc_spmm(xnl, xnr, idx_ne)
            hxsl, hxsr = _tc_merge(hxl, hxr, dep2)
            ol, orr = _sc_spmm(hxsl, hxsr, idx_en)
            xh, xnl, xnr = _tc_post(ol, orr, dv2, xh, w, bb, gg, be)
        return xh

    X1 = branch(h1_node_idx, h1_edge_idx, h1_DV_inv_sqrt, h1_DE_inv)
    X2 = branch(h2_node_idx, h2_edge_idx, h2_DV_inv_sqrt, h2_DE_inv)
    return _tc_fuse(X1, X2, Wa, ba.reshape(1, 1))[:N]


# speedup vs baseline: 1.2172x; 1.0127x over previous
import jax
import jax.numpy as jnp
from jax import lax
from jax.experimental import pallas as pl
from jax.experimental.pallas import tpu as pltpu
from jax.experimental.pallas import tpu_sc as plsc

N = 10000
D = 256
EH = 5000
NNZ = 160000
HD = 128
NT = 16
C = 128
N_IT = 80
NNZP = NT * N_IT * C
EHP = 5120
NODE_T = 632
NP = NT * NODE_T
ZR = 40
BN = 1264
BE = 632


def _zero_vmem(zb, nrows):
    def _zero_row(r, _):
        for v in range(HD // 16):
            zb[r, pl.ds(v * 16, 16)] = jnp.zeros((16,), jnp.float32)
        return _
    lax.fori_loop(0, nrows, _zero_row, None)


def _spmm_body(tabl, tabr, idx3, outl, outr,
               acc, islab, rows0, rows1,
               gsem0, gsem1, ssem, isem0, isem1, isem2, isem3):
    c = lax.axis_index("c")
    t = lax.axis_index("s")
    zb = rows0.at[pl.ds(0, ZR)]
    rem = NODE_T % ZR
    isems = [isem0, isem1, isem2, isem3]

    def _issue_islab(i, q):
        pltpu.async_copy(idx3.at[t, i], islab.at[q], isems[q])

    def _wait_islab(i, q):
        pltpu.make_async_copy(idx3.at[t, i], islab.at[q], isems[q]).wait()

    _issue_islab(0, 0)
    _issue_islab(1, 1)
    _issue_islab(2, 2)

    _zero_vmem(zb, ZR)
    for k in range(NODE_T // ZR):
        pltpu.async_copy(zb, acc.at[pl.ds(t * NODE_T + k * ZR, ZR)], ssem)
    pltpu.async_copy(rows0.at[pl.ds(0, rem)],
                     acc.at[pl.ds(t * NODE_T + NODE_T - rem, rem)], ssem)
    for k in range(NODE_T // ZR):
        pltpu.make_async_copy(
            zb, acc.at[pl.ds(t * NODE_T + k * ZR, ZR)], ssem).wait()
    pltpu.make_async_copy(
        rows0.at[pl.ds(0, rem)],
        acc.at[pl.ds(t * NODE_T + NODE_T - rem, rem)], ssem).wait()

    plsc.subcore_barrier()

    def _loop(tab):
        rb = [rows0, rows1]
        gs = [gsem0, gsem1]
        _wait_islab(0, 0)
        pltpu.async_copy(tab.at[islab.at[0, 0]], rows0, gsem0)

        @pl.loop(0, N_IT, step=4)
        def _(i):
            for u in range(4):
                s, o = u % 2, 1 - u % 2
                q, q1 = u % 4, (u + 1) % 4
                idx = i + u
                pltpu.make_async_copy(tab.at[islab.at[q, 0]], rb[s], gs[s]).wait()
                if u == 0:
                    @pl.when(i > 0)
                    def _(o=o):
                        pltpu.make_async_copy(
                            rb[o], acc.at[islab.at[3, 1]], ssem).wait()
                else:
                    pltpu.make_async_copy(
                        rb[o], acc.at[islab.at[q - 1, 1]], ssem).wait()
                if u < 3:
                    _wait_islab(idx + 1, q1)
                    pltpu.async_copy(tab.at[islab.at[q1, 0]], rb[o], gs[o])
                else:
                    @pl.when(idx + 1 < N_IT)
                    def _(idx=idx, o=o, q1=q1):
                        _wait_islab(idx + 1, q1)
                        pltpu.async_copy(tab.at[islab.at[q1, 0]], rb[o], gs[o])
                pltpu.async_copy(rb[s], acc.at[islab.at[q, 1]], ssem, add=True)
                @pl.when(idx + 3 < N_IT)
                def _(idx=idx, q=q):
                    _issue_islab(idx + 3, (q + 3) % 4)

        pltpu.make_async_copy(
            rb[(N_IT - 1) % 2], acc.at[islab.at[(N_IT - 1) % 4, 1]], ssem).wait()

    @pl.when(c == 0)
    def _():
        _loop(tabl)

    @pl.when(c == 1)
    def _():
        _loop(tabr)

    plsc.subcore_barrier()

    def _wout(out):
        rb = [rows0, rows1]
        ws = [gsem0, gsem1]
        nf = NODE_T // ZR
        base = t * NODE_T

        def _wr(b, off, nrows):
            return pltpu.make_async_copy(rb[b].at[pl.ds(0, nrows)],
                                         out.at[pl.ds(base + off, nrows)],
                                         ws[b])

        pltpu.sync_copy(acc.at[pl.ds(base, ZR)], rb[0].at[pl.ds(0, ZR)])
        for k in range(nf):
            b = k % 2
            if k >= 2:
                _wr(b, (k - 2) * ZR, ZR).wait()
            pltpu.async_copy(rb[b].at[pl.ds(0, ZR)],
                             out.at[pl.ds(base + k * ZR, ZR)], ws[b])
            if k + 1 < nf:
                pltpu.sync_copy(acc.at[pl.ds(base + (k + 1) * ZR, ZR)],
                                rb[1 - b].at[pl.ds(0, ZR)])
        bl = (nf - 1) % 2
        br = 1 - bl
        _wr(br, (nf - 2) * ZR, ZR).wait()
        pltpu.sync_copy(acc.at[pl.ds(base + nf * ZR, rem)],
                        rb[br].at[pl.ds(0, rem)])
        pltpu.async_copy(rb[br].at[pl.ds(0, rem)],
                         out.at[pl.ds(base + nf * ZR, rem)], ws[br])
        _wr(bl, (nf - 1) * ZR, ZR).wait()
        _wr(br, nf * ZR, rem).wait()

    @pl.when(c == 0)
    def _():
        _wout(outl)

    @pl.when(c == 1)
    def _():
        _wout(outr)


_sc_spmm = pl.kernel(
    _spmm_body,
    out_type=[jax.ShapeDtypeStruct((NP, HD), jnp.float32),
              jax.ShapeDtypeStruct((NP, HD), jnp.float32)],
    mesh=plsc.VectorSubcoreMesh(core_axis_name="c", subcore_axis_name="s"),
    scratch_types=[
        pltpu.VMEM_SHARED((NP, HD), jnp.float32),
        pltpu.VMEM((4, 2, C), jnp.int32),
        pltpu.VMEM((C, HD), jnp.float32),
        pltpu.VMEM((C, HD), jnp.float32),
        pltpu.SemaphoreType.DMA,
        pltpu.SemaphoreType.DMA,
        pltpu.SemaphoreType.DMA,
        pltpu.SemaphoreType.DMA,
        pltpu.SemaphoreType.DMA,
        pltpu.SemaphoreType.DMA,
        pltpu.SemaphoreType.DMA,
    ],
)


def _pre_body(x_ref, dv_ref, l_ref, r_ref):
    xn = x_ref[...] * dv_ref[...]
    l_ref[...] = xn[:, :HD]
    r_ref[...] = xn[:, HD:]


_tc_pre = pl.pallas_call(
    _pre_body,
    grid=(NP // BN,),
    in_specs=[pl.BlockSpec((BN, D), lambda i: (i, 0)),
              pl.BlockSpec((BN, 1), lambda i: (i, 0))],
    out_specs=[pl.BlockSpec((BN, HD), lambda i: (i, 0)),
               pl.BlockSpec((BN, HD), lambda i: (i, 0))],
    out_shape=[jax.ShapeDtypeStruct((NP, HD), jnp.float32),
               jax.ShapeDtypeStruct((NP, HD), jnp.float32)],
)


def _merge_body(a_ref, b_ref, de_ref, l_ref, r_ref):
    de = de_ref[...]
    l_ref[...] = a_ref[...] * de
    r_ref[...] = b_ref[...] * de


_tc_merge = pl.pallas_call(
    _merge_body,
    grid=(NP // BE,),
    in_specs=[pl.BlockSpec((BE, HD), lambda i: (i, 0)),
              pl.BlockSpec((BE, HD), lambda i: (i, 0)),
              pl.BlockSpec((BE, 1), lambda i: (i, 0))],
    out_specs=[pl.BlockSpec((BE, HD), lambda i: (i, 0)),
               pl.BlockSpec((BE, HD), lambda i: (i, 0))],
    out_shape=[jax.ShapeDtypeStruct((NP, HD), jnp.float32),
               jax.ShapeDtypeStruct((NP, HD), jnp.float32)],
)


def _post_body(l_ref, r_ref, dv_ref, res_ref, w_ref, b_ref, g_ref, be_ref,
               xh_ref, xnl_ref, xnr_ref):
    dv = dv_ref[...]
    xo = jnp.concatenate([l_ref[...], r_ref[...]], axis=1) * dv
    y = jnp.dot(xo, w_ref[...], preferred_element_type=jnp.float32)
    y = y + b_ref[...] + res_ref[...]
    mu = jnp.mean(y, axis=1, keepdims=True)
    yc = y - mu
    var = jnp.mean(yc * yc, axis=1, keepdims=True)
    z = yc * lax.rsqrt(var + 1e-5) * g_ref[...] + be_ref[...]
    xh = jnp.maximum(z, 0.0)
    xh_ref[...] = xh
    xn = xh * dv
    xnl_ref[...] = xn[:, :HD]
    xnr_ref[...] = xn[:, HD:]


_tc_post = pl.pallas_call(
    _post_body,
    grid=(NP // BN,),
    in_specs=[pl.BlockSpec((BN, HD), lambda i: (i, 0)),
              pl.BlockSpec((BN, HD), lambda i: (i, 0)),
              pl.BlockSpec((BN, 1), lambda i: (i, 0)),
              pl.BlockSpec((BN, D), lambda i: (i, 0)),
              pl.BlockSpec((D, D), lambda i: (0, 0)),
              pl.BlockSpec((1, D), lambda i: (0, 0)),
              pl.BlockSpec((1, D), lambda i: (0, 0)),
              pl.BlockSpec((1, D), lambda i: (0, 0))],
    out_specs=[pl.BlockSpec((BN, D), lambda i: (i, 0)),
               pl.BlockSpec((BN, HD), lambda i: (i, 0)),
               pl.BlockSpec((BN, HD), lambda i: (i, 0))],
    out_shape=[jax.ShapeDtypeStruct((NP, D), jnp.float32),
               jax.ShapeDtypeStruct((NP, HD), jnp.float32),
               jax.ShapeDtypeStruct((NP, HD), jnp.float32)],
)


def _fuse_body(x1_ref, x2_ref, wa_ref, ba_ref, o_ref):
    a = x1_ref[...]
    b = x2_ref[...]
    wv = wa_ref[...]
    s1 = jnp.dot(a, wv, preferred_element_type=jnp.float32) + ba_ref[...]
    s2 = jnp.dot(b, wv, preferred_element_type=jnp.float32) + ba_ref[...]
    m = jnp.maximum(s1, s2)
    e1 = jnp.exp(s1 - m)
    e2 = jnp.exp(s2 - m)
    w1 = e1 / (e1 + e2)
    o_ref[...] = w1 * a + (1.0 - w1) * b


_tc_fuse = pl.pallas_call(
    _fuse_body,
    grid=(NP // BN,),
    in_specs=[pl.BlockSpec((BN, D), lambda i: (i, 0)),
              pl.BlockSpec((BN, D), lambda i: (i, 0)),
              pl.BlockSpec((D, 1), lambda i: (0, 0)),
              pl.BlockSpec((1, 1), lambda i: (0, 0))],
    out_specs=pl.BlockSpec((BN, D), lambda i: (i, 0)),
    out_shape=jax.ShapeDtypeStruct((NP, D), jnp.float32),
)


def kernel(X, h1_node_idx, h1_edge_idx, h1_DV_inv_sqrt, h1_DE_inv,
           h2_node_idx, h2_edge_idx, h2_DV_inv_sqrt, h2_DE_inv,
           W1, b1, W2, b2, g1, beta1, g2, beta2, Wa, ba):
    params = [(W1, b1.reshape(1, D), g1.reshape(1, D), beta1.reshape(1, D)),
              (W2, b2.reshape(1, D), g2.reshape(1, D), beta2.reshape(1, D))]
    Xp = jnp.pad(X, ((0, NP - N), (0, 0)))

    def branch(nidx, eidx, dv, de):
        nidx = jnp.concatenate(
            [nidx.astype(jnp.int32), jnp.zeros((NNZP - NNZ,), jnp.int32)]
        ).reshape(NT, N_IT, C)
        eidx = jnp.concatenate(
            [eidx.astype(jnp.int32), jnp.full((NNZP - NNZ,), EHP - 1, jnp.int32)]
        ).reshape(NT, N_IT, C)
        idx_ne = jnp.stack([nidx, eidx], axis=2)
        idx_en = jnp.stack([eidx, nidx], axis=2)
        dv2 = jnp.pad(dv, (0, NP - N)).reshape(NP, 1)
        dep2 = jnp.pad(de, (0, NP - EH)).reshape(NP, 1)
        xh = Xp
        xnl, xnr = _tc_pre(Xp, dv2)
        for w, bb, gg, be in params:
            hxl, hxr = _sc_spmm(xnl, xnr, idx_ne)
            hxsl, hxsr = _tc_merge(hxl, hxr, dep2)
            ol, orr = _sc_spmm(hxsl, hxsr, idx_en)
            xh, xnl, xnr = _tc_post(ol, orr, dv2, xh, w, bb, gg, be)
        return xh

    X1 = branch(h1_node_idx, h1_edge_idx, h1_DV_inv_sqrt, h1_DE_inv)
    X2 = branch(h2_node_idx, h2_edge_idx, h2_DV_inv_sqrt, h2_DE_inv)
    return _tc_fuse(X1, X2, Wa, ba.reshape(1, 1))[:N]


# speedup vs baseline: 1.2501x; 1.0271x over previous
import jax
import jax.numpy as jnp
from jax import lax
from jax.experimental import pallas as pl
from jax.experimental.pallas import tpu as pltpu
from jax.experimental.pallas import tpu_sc as plsc

N = 10000
D = 256
EH = 5000
NNZ = 160000
HD = 128
NT = 16
C = 80
N_IT = 128
NNZP = NT * N_IT * C
EHP = 5120
NODE_T = 632
NP = NT * NODE_T
ZR = 40
BN = 1264
BE = 632


def _zero_vmem(zb, nrows):
    def _zero_row(r, _):
        for v in range(HD // 16):
            zb[r, pl.ds(v * 16, 16)] = jnp.zeros((16,), jnp.float32)
        return _
    lax.fori_loop(0, nrows, _zero_row, None)


def _spmm_body(tabl, tabr, idx3, outl, outr,
               acc, islab, rows0, rows1, rows2, rows3,
               gsem0, gsem1, gsem2, gsem3, ssem0, ssem1,
               isem0, isem1, isem2, isem3, isem4, isem5, isem6, isem7):
    c = lax.axis_index("c")
    t = lax.axis_index("s")
    zb = rows0.at[pl.ds(0, ZR)]
    rem = NODE_T % ZR
    rb = [rows0, rows1, rows2, rows3]
    gs = [gsem0, gsem1, gsem2, gsem3]
    ss = [ssem0, ssem1]
    isems = [isem0, isem1, isem2, isem3, isem4, isem5, isem6, isem7]

    def _issue_islab(i, q):
        pltpu.async_copy(idx3.at[t, i], islab.at[q], isems[q])

    def _wait_islab(i, q):
        pltpu.make_async_copy(idx3.at[t, i], islab.at[q], isems[q]).wait()

    for q in range(6):
        _issue_islab(q, q)

    _zero_vmem(zb, ZR)
    for k in range(NODE_T // ZR):
        pltpu.async_copy(zb, acc.at[pl.ds(t * NODE_T + k * ZR, ZR)], ssem0)
    pltpu.async_copy(rows0.at[pl.ds(0, rem)],
                     acc.at[pl.ds(t * NODE_T + NODE_T - rem, rem)], ssem0)
    for k in range(NODE_T // ZR):
        pltpu.make_async_copy(
            zb, acc.at[pl.ds(t * NODE_T + k * ZR, ZR)], ssem0).wait()
    pltpu.make_async_copy(
        rows0.at[pl.ds(0, rem)],
        acc.at[pl.ds(t * NODE_T + NODE_T - rem, rem)], ssem0).wait()

    plsc.subcore_barrier()

    def _loop(tab):
        _wait_islab(0, 0)
        pltpu.async_copy(tab.at[islab.at[0, 0]], rb[0], gs[0])
        _wait_islab(1, 1)
        pltpu.async_copy(tab.at[islab.at[1, 0]], rb[1], gs[1])

        @pl.loop(0, N_IT, step=8)
        def _(i):
            for u in range(8):
                idx = i + u
                r = u % 4
                p = u % 2
                pltpu.make_async_copy(tab.at[islab.at[u % 8, 0]],
                                      rb[r], gs[r]).wait()
                if u >= 2:
                    pltpu.make_async_copy(
                        rb[(u - 2) % 4], acc.at[islab.at[u - 2, 1]],
                        ss[p]).wait()
                else:
                    @pl.when(i > 0)
                    def _(u=u, p=p):
                        pltpu.make_async_copy(
                            rb[u + 2], acc.at[islab.at[u + 6, 1]], ss[p]).wait()
                if u < 6:
                    _wait_islab(idx + 2, u + 2)
                    pltpu.async_copy(tab.at[islab.at[u + 2, 0]],
                                     rb[(u + 2) % 4], gs[(u + 2) % 4])
                else:
                    @pl.when(idx + 2 < N_IT)
                    def _(idx=idx, u=u):
                        _wait_islab(idx + 2, (u + 2) % 8)
                        pltpu.async_copy(tab.at[islab.at[(u + 2) % 8, 0]],
                                         rb[(u + 2) % 4], gs[(u + 2) % 4])
                pltpu.async_copy(rb[r], acc.at[islab.at[u % 8, 1]], ss[p],
                                 add=True)
                if u < 2:
                    _issue_islab(idx + 6, u + 6)
                else:
                    @pl.when(idx + 6 < N_IT)
                    def _(idx=idx, u=u):
                        _issue_islab(idx + 6, (u + 6) % 8)

        pltpu.make_async_copy(
            rb[(N_IT - 2) % 4], acc.at[islab.at[(N_IT - 2) % 8, 1]],
            ss[(N_IT - 2) % 2]).wait()
        pltpu.make_async_copy(
            rb[(N_IT - 1) % 4], acc.at[islab.at[(N_IT - 1) % 8, 1]],
            ss[(N_IT - 1) % 2]).wait()

    @pl.when(c == 0)
    def _():
        _loop(tabl)

    @pl.when(c == 1)
    def _():
        _loop(tabr)

    plsc.subcore_barrier()

    def _wout(out):
        rb = [rows0, rows1]
        ws = [gsem0, gsem1]
        nf = NODE_T // ZR
        base = t * NODE_T

        def _wr(b, off, nrows):
            return pltpu.make_async_copy(rb[b].at[pl.ds(0, nrows)],
                                         out.at[pl.ds(base + off, nrows)],
                                         ws[b])

        pltpu.sync_copy(acc.at[pl.ds(base, ZR)], rb[0].at[pl.ds(0, ZR)])
        for k in range(nf):
            b = k % 2
            if k >= 2:
                _wr(b, (k - 2) * ZR, ZR).wait()
            pltpu.async_copy(rb[b].at[pl.ds(0, ZR)],
                             out.at[pl.ds(base + k * ZR, ZR)], ws[b])
            if k + 1 < nf:
                pltpu.sync_copy(acc.at[pl.ds(base + (k + 1) * ZR, ZR)],
                                rb[1 - b].at[pl.ds(0, ZR)])
        bl = (nf - 1) % 2
        br = 1 - bl
        _wr(br, (nf - 2) * ZR, ZR).wait()
        pltpu.sync_copy(acc.at[pl.ds(base + nf * ZR, rem)],
                        rb[br].at[pl.ds(0, rem)])
        pltpu.async_copy(rb[br].at[pl.ds(0, rem)],
                         out.at[pl.ds(base + nf * ZR, rem)], ws[br])
        _wr(bl, (nf - 1) * ZR, ZR).wait()
        _wr(br, nf * ZR, rem).wait()

    @pl.when(c == 0)
    def _():
        _wout(outl)

    @pl.when(c == 1)
    def _():
        _wout(outr)


_sc_spmm = pl.kernel(
    _spmm_body,
    out_type=[jax.ShapeDtypeStruct((NP, HD), jnp.float32),
              jax.ShapeDtypeStruct((NP, HD), jnp.float32)],
    mesh=plsc.VectorSubcoreMesh(core_axis_name="c", subcore_axis_name="s"),
    scratch_types=(
        [pltpu.VMEM_SHARED((NP, HD), jnp.float32),
         pltpu.VMEM((8, 2, C), jnp.int32)]
        + [pltpu.VMEM((C, HD), jnp.float32)] * 4
        + [pltpu.SemaphoreType.DMA] * 14
    ),
)


def _pre_body(x_ref, dv_ref, l_ref, r_ref):
    xn = x_ref[...] * dv_ref[...]
    l_ref[...] = xn[:, :HD]
    r_ref[...] = xn[:, HD:]


_tc_pre = pl.pallas_call(
    _pre_body,
    grid=(NP // BN,),
    in_specs=[pl.BlockSpec((BN, D), lambda i: (i, 0)),
              pl.BlockSpec((BN, 1), lambda i: (i, 0))],
    out_specs=[pl.BlockSpec((BN, HD), lambda i: (i, 0)),
               pl.BlockSpec((BN, HD), lambda i: (i, 0))],
    out_shape=[jax.ShapeDtypeStruct((NP, HD), jnp.float32),
               jax.ShapeDtypeStruct((NP, HD), jnp.float32)],
)


def _merge_body(a_ref, b_ref, de_ref, l_ref, r_ref):
    de = de_ref[...]
    l_ref[...] = a_ref[...] * de
    r_ref[...] = b_ref[...] * de


_tc_merge = pl.pallas_call(
    _merge_body,
    grid=(NP // BE,),
    in_specs=[pl.BlockSpec((BE, HD), lambda i: (i, 0)),
              pl.BlockSpec((BE, HD), lambda i: (i, 0)),
              pl.BlockSpec((BE, 1), lambda i: (i, 0))],
    out_specs=[pl.BlockSpec((BE, HD), lambda i: (i, 0)),
               pl.BlockSpec((BE, HD), lambda i: (i, 0))],
    out_shape=[jax.ShapeDtypeStruct((NP, HD), jnp.float32),
               jax.ShapeDtypeStruct((NP, HD), jnp.float32)],
)


def _post_body(l_ref, r_ref, dv_ref, res_ref, w_ref, b_ref, g_ref, be_ref,
               xh_ref, xnl_ref, xnr_ref):
    dv = dv_ref[...]
    xo = jnp.concatenate([l_ref[...], r_ref[...]], axis=1) * dv
    y = jnp.dot(xo, w_ref[...], preferred_element_type=jnp.float32)
    y = y + b_ref[...] + res_ref[...]
    mu = jnp.mean(y, axis=1, keepdims=True)
    yc = y - mu
    var = jnp.mean(yc * yc, axis=1, keepdims=True)
    z = yc * lax.rsqrt(var + 1e-5) * g_ref[...] + be_ref[...]
    xh = jnp.maximum(z, 0.0)
    xh_ref[...] = xh
    xn = xh * dv
    xnl_ref[...] = xn[:, :HD]
    xnr_ref[...] = xn[:, HD:]


_tc_post = pl.pallas_call(
    _post_body,
    grid=(NP // BN,),
    in_specs=[pl.BlockSpec((BN, HD), lambda i: (i, 0)),
              pl.BlockSpec((BN, HD), lambda i: (i, 0)),
              pl.BlockSpec((BN, 1), lambda i: (i, 0)),
              pl.BlockSpec((BN, D), lambda i: (i, 0)),
              pl.BlockSpec((D, D), lambda i: (0, 0)),
              pl.BlockSpec((1, D), lambda i: (0, 0)),
              pl.BlockSpec((1, D), lambda i: (0, 0)),
              pl.BlockSpec((1, D), lambda i: (0, 0))],
    out_specs=[pl.BlockSpec((BN, D), lambda i: (i, 0)),
               pl.BlockSpec((BN, HD), lambda i: (i, 0)),
               pl.BlockSpec((BN, HD), lambda i: (i, 0))],
    out_shape=[jax.ShapeDtypeStruct((NP, D), jnp.float32),
               jax.ShapeDtypeStruct((NP, HD), jnp.float32),
               jax.ShapeDtypeStruct((NP, HD), jnp.float32)],
)


def _fuse_body(x1_ref, x2_ref, wa_ref, ba_ref, o_ref):
    a = x1_ref[...]
    b = x2_ref[...]
    wv = wa_ref[...]
    s1 = jnp.dot(a, wv, preferred_element_type=jnp.float32) + ba_ref[...]
    s2 = jnp.dot(b, wv, preferred_element_type=jnp.float32) + ba_ref[...]
    m = jnp.maximum(s1, s2)
    e1 = jnp.exp(s1 - m)
    e2 = jnp.exp(s2 - m)
    w1 = e1 / (e1 + e2)
    o_ref[...] = w1 * a + (1.0 - w1) * b


_tc_fuse = pl.pallas_call(
    _fuse_body,
    grid=(NP // BN,),
    in_specs=[pl.BlockSpec((BN, D), lambda i: (i, 0)),
              pl.BlockSpec((BN, D), lambda i: (i, 0)),
              pl.BlockSpec((D, 1), lambda i: (0, 0)),
              pl.BlockSpec((1, 1), lambda i: (0, 0))],
    out_specs=pl.BlockSpec((BN, D), lambda i: (i, 0)),
    out_shape=jax.ShapeDtypeStruct((NP, D), jnp.float32),
)


def kernel(X, h1_node_idx, h1_edge_idx, h1_DV_inv_sqrt, h1_DE_inv,
           h2_node_idx, h2_edge_idx, h2_DV_inv_sqrt, h2_DE_inv,
           W1, b1, W2, b2, g1, beta1, g2, beta2, Wa, ba):
    params = [(W1, b1.reshape(1, D), g1.reshape(1, D), beta1.reshape(1, D)),
              (W2, b2.reshape(1, D), g2.reshape(1, D), beta2.reshape(1, D))]
    Xp = jnp.pad(X, ((0, NP - N), (0, 0)))

    def branch(nidx, eidx, dv, de):
        nidx = jnp.concatenate(
            [nidx.astype(jnp.int32), jnp.zeros((NNZP - NNZ,), jnp.int32)]
        ).reshape(NT, N_IT, C)
        eidx = jnp.concatenate(
            [eidx.astype(jnp.int32), jnp.full((NNZP - NNZ,), EHP - 1, jnp.int32)]
        ).reshape(NT, N_IT, C)
        idx_ne = jnp.stack([nidx, eidx], axis=2)
        idx_en = jnp.stack([eidx, nidx], axis=2)
        dv2 = jnp.pad(dv, (0, NP - N)).reshape(NP, 1)
        dep2 = jnp.pad(de, (0, NP - EH)).reshape(NP, 1)
        xh = Xp
        xnl, xnr = _tc_pre(Xp, dv2)
        for w, bb, gg, be in params:
            hxl, hxr = _sc_spmm(xnl, xnr, idx_ne)
            hxsl, hxsr = _tc_merge(hxl, hxr, dep2)
            ol, orr = _sc_spmm(hxsl, hxsr, idx_en)
            xh, xnl, xnr = _tc_post(ol, orr, dv2, xh, w, bb, gg, be)
        return xh

    X1 = branch(h1_node_idx, h1_edge_idx, h1_DV_inv_sqrt, h1_DE_inv)
    X2 = branch(h2_node_idx, h2_edge_idx, h2_DV_inv_sqrt, h2_DE_inv)
    return _tc_fuse(X1, X2, Wa, ba.reshape(1, 1))[:N]
